# baseline (speedup 1.0000x reference)
import functools

import jax
import jax.numpy as jnp
from jax import lax
from jax.experimental import pallas as pl
from jax.experimental.pallas import tpu as pltpu
from jax.experimental.pallas import tpu_sc as plsc

_NUM_USERS = 1000000
_EMBED_DIM = 64
_BATCH = 16384


def _make_gather(batch, dim):
    info = plsc.get_sparse_core_info()
    nc, ns = info.num_cores, info.num_subcores
    nw = nc * ns
    assert batch % (8 * nw) == 0
    b_per_w = batch // nw
    mesh = plsc.VectorSubcoreMesh(core_axis_name="c", subcore_axis_name="s")

    @functools.partial(
        pl.kernel,
        mesh=mesh,
        out_type=jax.ShapeDtypeStruct((batch, dim), jnp.float32),
        scratch_types=[
            pltpu.VMEM((b_per_w,), jnp.int32),
            pltpu.VMEM((b_per_w, dim), jnp.float32),
            pltpu.SemaphoreType.DMA,
        ],
        compiler_params=pltpu.CompilerParams(use_tc_tiling_on_sc=False),
    )
    def gather_kernel(idx_hbm, table_hbm, out_hbm, idx_v, rows_v, sem):
        wid = lax.axis_index("s") * nc + lax.axis_index("c")
        base = wid * b_per_w
        pltpu.sync_copy(idx_hbm.at[pl.ds(base, b_per_w)], idx_v)
        pltpu.async_copy(table_hbm.at[idx_v], rows_v, sem).wait()
        pltpu.sync_copy(rows_v, out_hbm.at[pl.ds(base, b_per_w)])

    return gather_kernel


def kernel(user_idx, table):
    return _make_gather(_BATCH, _EMBED_DIM)(user_idx.astype(jnp.int32), table)


# speedup vs baseline: 2.1773x; 2.1773x over previous
import functools

import jax
import jax.numpy as jnp
from jax import lax
from jax.experimental import pallas as pl
from jax.experimental.pallas import tpu as pltpu
from jax.experimental.pallas import tpu_sc as plsc

_NUM_USERS = 1000000
_EMBED_DIM = 64
_BATCH = 16384
_RPB = 8
_CH = 32
_LANES = 16


def _make_gather(batch, dim):
    info = plsc.get_sparse_core_info()
    nc, ns = info.num_cores, info.num_subcores
    nw = nc * ns
    b_per_w = batch // nw
    nch = b_per_w // _CH
    obpc = _CH // _RPB
    mesh = plsc.VectorSubcoreMesh(core_axis_name="c", subcore_axis_name="s")

    @functools.partial(
        pl.kernel,
        mesh=mesh,
        out_type=jax.ShapeDtypeStruct((batch // _RPB, _RPB, dim), jnp.float32),
        scratch_types=[
            pltpu.VMEM((b_per_w,), jnp.int32),
            pltpu.VMEM((b_per_w,), jnp.int32),
            pltpu.VMEM((_CH, _RPB, dim), jnp.float32),
            pltpu.VMEM((_CH, _RPB, dim), jnp.float32),
            pltpu.VMEM((obpc, _RPB, dim), jnp.float32),
            pltpu.SemaphoreType.DMA,
            pltpu.SemaphoreType.DMA,
        ],
        compiler_params=pltpu.CompilerParams(needs_layout_passes=False),
    )
    def gather_kernel(idx_hbm, tab_hbm, out_hbm, idx_v, blk_v, buf0, buf1,
                      out_v, sem0, sem1):
        wid = lax.axis_index("s") * nc + lax.axis_index("c")
        base = wid * b_per_w
        bufs = (buf0, buf1)
        sems = (sem0, sem1)

        pltpu.sync_copy(idx_hbm.at[pl.ds(base, b_per_w)], idx_v)
        for s in range(b_per_w // _LANES):
            sl = pl.ds(s * _LANES, _LANES)
            blk_v[sl] = idx_v[sl] >> 3

        lane = lax.iota(jnp.int32, _LANES)

        def start(g, b):
            for h in range(_CH // _LANES):
                blks = blk_v[pl.ds(g * _CH + h * _LANES, _LANES)]
                for l in range(_LANES):
                    s = jnp.sum(jnp.where(lane == l, blks, 0))
                    j = h * _LANES + l
                    pltpu.async_copy(tab_hbm.at[s], bufs[b].at[j], sems[b])

        def extract(g, b):
            lane = lax.iota(jnp.int32, _LANES)
            for h in range(_CH // _LANES):
                rems = idx_v[pl.ds(g * _CH + h * _LANES, _LANES)] & 7
                src0 = lane + h * _LANES
                jj = src0
                dst0 = jj >> 3
                dst1 = jj & 7
                for c in range(dim):
                    col = jnp.full((_LANES,), c, jnp.int32)
                    x = plsc.load_gather(bufs[b], [src0, rems, col])
                    plsc.store_scatter(out_v, [dst0, dst1, col], x)
            pltpu.sync_copy(
                out_v, out_hbm.at[pl.ds(wid * (b_per_w // _RPB) + g * obpc,
                                        obpc)])

        start(0, 0)

        def step(i, carry):
            g = i * 2
            for b in range(2):
                gg = g + b
                pltpu.make_async_copy(
                    tab_hbm.at[pl.ds(0, _CH)], bufs[b], sems[b]).wait()

                @pl.when(gg + 1 < nch)
                def _():
                    start(gg + 1, 1 - b)

                extract(gg, b)
            return carry

        lax.fori_loop(0, nch // 2, step, 0)

    return gather_kernel


def kernel(user_idx, table):
    tab3 = table.reshape(_NUM_USERS // _RPB, _RPB, _EMBED_DIM)
    out3 = _make_gather(_BATCH, _EMBED_DIM)(user_idx.astype(jnp.int32), tab3)
    return out3.reshape(_BATCH, _EMBED_DIM)
